# baseline (speedup 1.0000x reference)
import functools

import jax
import jax.numpy as jnp
from jax import lax
from jax.experimental import pallas as pl
from jax.experimental.pallas import tpu as pltpu
from jax.experimental.pallas import tpu_sc as plsc

NC = 2
NS = 16
L = 16
NW = NC * NS

B = 16384
D = 64
W = 128
BPW = B // NW
CH = 128
NCH = BPW // CH
GPC = CH // L


def _mf_body(uidx_hbm, iidx_hbm, utab_hbm, itab_hbm, ubias_hbm, ibias_hbm,
             out_hbm, uidx_v, iidx_v, upair_v, ipair_v, urows_v, irows_v,
             ub_v, ib_v, res_v, sem, bsem):
    wid = lax.axis_index("s") * NC + lax.axis_index("c")

    pltpu.sync_copy(uidx_hbm.at[wid], uidx_v)
    pltpu.sync_copy(iidx_hbm.at[wid], iidx_v)

    for c in range(NCH):
        for j in range(CH // L):
            sl = pl.ds(j * L, L)
            upair_v[c, sl] = uidx_v[c, sl] >> 1
            ipair_v[c, sl] = iidx_v[c, sl] >> 1

    lane = lax.iota(jnp.int32, L)

    for c in range(NCH):
        sl = pl.ds(c * CH, CH)
        pend = [
            pltpu.async_copy(utab_hbm.at[upair_v.at[c]], urows_v, sem),
            pltpu.async_copy(itab_hbm.at[ipair_v.at[c]], irows_v, sem),
            pltpu.async_copy(ubias_hbm.at[uidx_v.at[c]], ub_v, bsem),
            pltpu.async_copy(ibias_hbm.at[iidx_v.at[c]], ib_v, bsem),
        ]
        for p in pend:
            p.wait()

        def gbody(g, carry):
            rows = g * L + lane
            gsl = pl.ds(g * L, L)
            uoff = (uidx_v[c, gsl] & 1) << 6
            ioff = (iidx_v[c, gsl] & 1) << 6
            acc = ub_v[gsl] + ib_v[gsl]
            for k in range(D):
                u = plsc.load_gather(urows_v, [rows, uoff + k])
                v = plsc.load_gather(irows_v, [rows, ioff + k])
                acc = acc + u * v
            res_v[pl.ds(c * CH + g * L, L)] = acc
            return carry

        lax.fori_loop(0, GPC, gbody, 0)

    pltpu.sync_copy(res_v, out_hbm.at[wid])


@jax.jit
def _mf(user_indices, item_indices, user_embedding, item_embedding,
        user_bias, item_bias):
    uidx = user_indices.astype(jnp.int32).reshape(NW, NCH, CH)
    iidx = item_indices.astype(jnp.int32).reshape(NW, NCH, CH)
    ut = user_embedding.reshape(-1, W)
    it = item_embedding.reshape(-1, W)
    ub = user_bias.reshape(-1)
    ib = item_bias.reshape(-1)

    mesh = plsc.VectorSubcoreMesh(core_axis_name="c", subcore_axis_name="s")
    run = pl.kernel(
        _mf_body,
        out_type=jax.ShapeDtypeStruct((NW, BPW), jnp.float32),
        mesh=mesh,
        compiler_params=pltpu.CompilerParams(
            needs_layout_passes=False, use_tc_tiling_on_sc=True),
        scratch_types=[
            pltpu.VMEM((NCH, CH), jnp.int32),
            pltpu.VMEM((NCH, CH), jnp.int32),
            pltpu.VMEM((NCH, CH), jnp.int32),
            pltpu.VMEM((NCH, CH), jnp.int32),
            pltpu.VMEM((CH, W), jnp.float32),
            pltpu.VMEM((CH, W), jnp.float32),
            pltpu.VMEM((CH,), jnp.float32),
            pltpu.VMEM((CH,), jnp.float32),
            pltpu.VMEM((BPW,), jnp.float32),
            pltpu.SemaphoreType.DMA,
            pltpu.SemaphoreType.DMA,
        ],
    )
    out = run(uidx, iidx, ut, it, ub, ib)
    return out.reshape(B)


def kernel(user_indices, item_indices, user_embedding, item_embedding,
           user_bias, item_bias):
    return _mf(user_indices, item_indices, user_embedding, item_embedding,
               user_bias, item_bias)


# speedup vs baseline: 1.0055x; 1.0055x over previous
import functools

import jax
import jax.numpy as jnp
from jax import lax
from jax.experimental import pallas as pl
from jax.experimental.pallas import tpu as pltpu
from jax.experimental.pallas import tpu_sc as plsc

NC = 2
NS = 16
L = 16
NW = NC * NS

B = 16384
D = 64
W = 128
BPW = B // NW
CH = 128
NCH = BPW // CH
GPC = CH // L


def _mf_body(uidx_hbm, iidx_hbm, utab_hbm, itab_hbm, ubias_hbm, ibias_hbm,
             out_hbm, uidx_v, iidx_v, upair_v, ipair_v, urows_v, irows_v,
             ub_v, ib_v, res_v, sem0, sem1, bsem0, bsem1):
    wid = lax.axis_index("s") * NC + lax.axis_index("c")

    pltpu.sync_copy(uidx_hbm.at[wid], uidx_v)
    pltpu.sync_copy(iidx_hbm.at[wid], iidx_v)

    for c in range(NCH):
        for j in range(GPC):
            sl = pl.ds(j * L, L)
            upair_v[c, sl] = uidx_v[c, sl] >> 1
            ipair_v[c, sl] = iidx_v[c, sl] >> 1

    sems = (sem0, sem1)
    bsems = (bsem0, bsem1)

    def fire(c):
        s = c % 2
        return [
            pltpu.async_copy(utab_hbm.at[upair_v.at[c]], urows_v.at[s], sems[s]),
            pltpu.async_copy(itab_hbm.at[ipair_v.at[c]], irows_v.at[s], sems[s]),
            pltpu.async_copy(ubias_hbm.at[uidx_v.at[c]], ub_v.at[s], bsems[s]),
            pltpu.async_copy(ibias_hbm.at[iidx_v.at[c]], ib_v.at[s], bsems[s]),
        ]

    lane = lax.iota(jnp.int32, L)
    pend = {0: fire(0)}
    for c in range(NCH):
        s = c % 2
        if c + 1 < NCH:
            pend[c + 1] = fire(c + 1)
        for p in pend.pop(c):
            p.wait()

        def gbody(g, carry):
            rows = g * L + lane
            gsl = pl.ds(g * L, L)
            uoff = (uidx_v[c, gsl] & 1) << 6
            ioff = (iidx_v[c, gsl] & 1) << 6
            acc = ub_v[s, gsl] + ib_v[s, gsl]
            for k in range(D):
                u = plsc.load_gather(urows_v.at[s], [rows, uoff + k])
                v = plsc.load_gather(irows_v.at[s], [rows, ioff + k])
                acc = acc + u * v
            res_v[pl.ds(c * CH + g * L, L)] = acc
            return carry

        lax.fori_loop(0, GPC, gbody, 0)

    pltpu.sync_copy(res_v, out_hbm.at[wid])


@jax.jit
def _mf(user_indices, item_indices, user_embedding, item_embedding,
        user_bias, item_bias):
    uidx = user_indices.astype(jnp.int32).reshape(NW, NCH, CH)
    iidx = item_indices.astype(jnp.int32).reshape(NW, NCH, CH)
    ut = user_embedding.reshape(-1, W)
    it = item_embedding.reshape(-1, W)
    ub = user_bias.reshape(-1)
    ib = item_bias.reshape(-1)

    mesh = plsc.VectorSubcoreMesh(core_axis_name="c", subcore_axis_name="s")
    run = pl.kernel(
        _mf_body,
        out_type=jax.ShapeDtypeStruct((NW, BPW), jnp.float32),
        mesh=mesh,
        compiler_params=pltpu.CompilerParams(
            needs_layout_passes=False, use_tc_tiling_on_sc=True),
        scratch_types=[
            pltpu.VMEM((NCH, CH), jnp.int32),
            pltpu.VMEM((NCH, CH), jnp.int32),
            pltpu.VMEM((NCH, CH), jnp.int32),
            pltpu.VMEM((NCH, CH), jnp.int32),
            pltpu.VMEM((2, CH, W), jnp.float32),
            pltpu.VMEM((2, CH, W), jnp.float32),
            pltpu.VMEM((2, CH), jnp.float32),
            pltpu.VMEM((2, CH), jnp.float32),
            pltpu.VMEM((BPW,), jnp.float32),
            pltpu.SemaphoreType.DMA,
            pltpu.SemaphoreType.DMA,
            pltpu.SemaphoreType.DMA,
            pltpu.SemaphoreType.DMA,
        ],
    )
    out = run(uidx, iidx, ut, it, ub, ib)
    return out.reshape(B)


def kernel(user_indices, item_indices, user_embedding, item_embedding,
           user_bias, item_bias):
    return _mf(user_indices, item_indices, user_embedding, item_embedding,
               user_bias, item_bias)


# speedup vs baseline: 2.2067x; 2.1946x over previous
import functools

import jax
import jax.numpy as jnp
from jax import lax
from jax.experimental import pallas as pl
from jax.experimental.pallas import tpu as pltpu
from jax.experimental.pallas import tpu_sc as plsc

NC = 2
NS = 16
L = 16
NW = NC * NS

B = 16384
D = 64
BPW = B // NW
CH = 16
NCH = BPW // CH
GPC = CH // L


def _mf_body(uidx_hbm, iidx_hbm, utab_hbm, itab_hbm, ubias_hbm, ibias_hbm,
             out_hbm, uidx_v, iidx_v, ublk_v, iblk_v, ubuf_v, ibuf_v,
             ub_v, ib_v, res_v, sem0, sem1, bsem0, bsem1):
    wid = lax.axis_index("s") * NC + lax.axis_index("c")

    pltpu.sync_copy(uidx_hbm.at[wid], uidx_v)
    pltpu.sync_copy(iidx_hbm.at[wid], iidx_v)

    for c in range(NCH):
        for j in range(GPC):
            sl = pl.ds(j * L, L)
            ublk_v[c, sl] = uidx_v[c, sl] >> 3
            iblk_v[c, sl] = iidx_v[c, sl] >> 3

    sems = (sem0, sem1)
    bsems = (bsem0, bsem1)

    lane = lax.iota(jnp.int32, L)
    sl16 = pl.ds(0, L)

    def fire(c, s):
        @pl.when(c < NCH)
        def _():
            uvec = ublk_v[c, sl16]
            ivec = iblk_v[c, sl16]
            for l in range(L):
                pltpu.async_copy(utab_hbm.at[uvec[l]],
                                 ubuf_v.at[s].at[l], sems[s])
                pltpu.async_copy(itab_hbm.at[ivec[l]],
                                 ibuf_v.at[s].at[l], sems[s])
            pltpu.async_copy(ubias_hbm.at[uidx_v.at[c]], ub_v.at[s], bsems[s])
            pltpu.async_copy(ibias_hbm.at[iidx_v.at[c]], ib_v.at[s], bsems[s])

    def drain(s):
        pltpu.make_async_copy(utab_hbm.at[pl.ds(0, CH)], ubuf_v.at[s],
                              sems[s]).wait()
        pltpu.make_async_copy(itab_hbm.at[pl.ds(0, CH)], ibuf_v.at[s],
                              sems[s]).wait()
        pltpu.make_async_copy(ubias_hbm.at[pl.ds(0, CH)], ub_v.at[s],
                              bsems[s]).wait()
        pltpu.make_async_copy(ibias_hbm.at[pl.ds(0, CH)], ib_v.at[s],
                              bsems[s]).wait()

    def compute(c, s):
        usub = uidx_v[c, sl16] & 7
        isub = iidx_v[c, sl16] & 7
        acc = ub_v[s, sl16] + ib_v[s, sl16]
        for k in range(D):
            kk = jnp.full((L,), k, jnp.int32)
            u = plsc.load_gather(ubuf_v.at[s], [lane, usub, kk])
            v = plsc.load_gather(ibuf_v.at[s], [lane, isub, kk])
            acc = acc + u * v
        res_v[pl.ds(c * CH, L)] = acc

    fire(0, 0)
    fire(1, 1)

    def tbody(t, carry):
        c0 = 2 * t
        drain(0)
        compute(c0, 0)
        fire(c0 + 2, 0)
        drain(1)
        compute(c0 + 1, 1)
        fire(c0 + 3, 1)
        return carry

    lax.fori_loop(0, NCH // 2, tbody, 0)

    pltpu.sync_copy(res_v, out_hbm.at[wid])


@jax.jit
def _mf(user_indices, item_indices, user_embedding, item_embedding,
        user_bias, item_bias):
    uidx = user_indices.astype(jnp.int32).reshape(NW, NCH, CH)
    iidx = item_indices.astype(jnp.int32).reshape(NW, NCH, CH)
    ut = user_embedding.reshape(-1, 8, D)
    it = item_embedding.reshape(-1, 8, D)
    ub = user_bias.reshape(-1)
    ib = item_bias.reshape(-1)

    mesh = plsc.VectorSubcoreMesh(core_axis_name="c", subcore_axis_name="s")
    run = pl.kernel(
        _mf_body,
        out_type=jax.ShapeDtypeStruct((NW, BPW), jnp.float32),
        mesh=mesh,
        compiler_params=pltpu.CompilerParams(
            needs_layout_passes=False, use_tc_tiling_on_sc=True),
        scratch_types=[
            pltpu.VMEM((NCH, CH), jnp.int32),
            pltpu.VMEM((NCH, CH), jnp.int32),
            pltpu.VMEM((NCH, CH), jnp.int32),
            pltpu.VMEM((NCH, CH), jnp.int32),
            pltpu.VMEM((2, CH, 8, D), jnp.float32),
            pltpu.VMEM((2, CH, 8, D), jnp.float32),
            pltpu.VMEM((2, CH), jnp.float32),
            pltpu.VMEM((2, CH), jnp.float32),
            pltpu.VMEM((BPW,), jnp.float32),
            pltpu.SemaphoreType.DMA,
            pltpu.SemaphoreType.DMA,
            pltpu.SemaphoreType.DMA,
            pltpu.SemaphoreType.DMA,
        ],
    )
    out = run(uidx, iidx, ut, it, ub, ib)
    return out.reshape(B)


def kernel(user_indices, item_indices, user_embedding, item_embedding,
           user_bias, item_bias):
    return _mf(user_indices, item_indices, user_embedding, item_embedding,
               user_bias, item_bias)


# speedup vs baseline: 2.2388x; 1.0146x over previous
import functools

import jax
import jax.numpy as jnp
from jax import lax
from jax.experimental import pallas as pl
from jax.experimental.pallas import tpu as pltpu
from jax.experimental.pallas import tpu_sc as plsc

NC = 2
NS = 16
L = 16
NW = NC * NS

B = 16384
D = 64
BPW = B // NW
CH = 16
NCH = BPW // CH
GPC = CH // L


def _mf_body(uidx_hbm, iidx_hbm, utab_hbm, itab_hbm, ubias_hbm, ibias_hbm,
             out_hbm, uidx_v, iidx_v, ublk_v, iblk_v, ubuf_v, ibuf_v,
             ub_v, ib_v, res_v, sem0, sem1, sem2, bsem0, bsem1, bsem2):
    wid = lax.axis_index("s") * NC + lax.axis_index("c")

    pltpu.sync_copy(uidx_hbm.at[wid], uidx_v)
    pltpu.sync_copy(iidx_hbm.at[wid], iidx_v)

    for c in range(NCH):
        for j in range(GPC):
            sl = pl.ds(j * L, L)
            ublk_v[c, sl] = uidx_v[c, sl] >> 3
            iblk_v[c, sl] = iidx_v[c, sl] >> 3

    sems = (sem0, sem1, sem2)
    bsems = (bsem0, bsem1, bsem2)

    lane = lax.iota(jnp.int32, L)
    sl16 = pl.ds(0, L)

    def fire(c, s):
        @pl.when(c < NCH)
        def _():
            uvec = ublk_v[c, sl16]
            ivec = iblk_v[c, sl16]
            for l in range(L):
                pltpu.async_copy(utab_hbm.at[uvec[l]],
                                 ubuf_v.at[s].at[l], sems[s])
                pltpu.async_copy(itab_hbm.at[ivec[l]],
                                 ibuf_v.at[s].at[l], sems[s])
            pltpu.async_copy(ubias_hbm.at[uidx_v.at[c]], ub_v.at[s], bsems[s])
            pltpu.async_copy(ibias_hbm.at[iidx_v.at[c]], ib_v.at[s], bsems[s])

    def drain(s):
        pltpu.make_async_copy(utab_hbm.at[pl.ds(0, CH)], ubuf_v.at[s],
                              sems[s]).wait()
        pltpu.make_async_copy(itab_hbm.at[pl.ds(0, CH)], ibuf_v.at[s],
                              sems[s]).wait()
        pltpu.make_async_copy(ubias_hbm.at[pl.ds(0, CH)], ub_v.at[s],
                              bsems[s]).wait()
        pltpu.make_async_copy(ibias_hbm.at[pl.ds(0, CH)], ib_v.at[s],
                              bsems[s]).wait()

    def compute(c, s):
        usub = uidx_v[c, sl16] & 7
        isub = iidx_v[c, sl16] & 7
        acc = ub_v[s, sl16] + ib_v[s, sl16]
        for k in range(D):
            kk = jnp.full((L,), k, jnp.int32)
            u = plsc.load_gather(ubuf_v.at[s], [lane, usub, kk])
            v = plsc.load_gather(ibuf_v.at[s], [lane, isub, kk])
            acc = acc + u * v
        res_v[pl.ds(c * CH, L)] = acc

    NSLOT = 3
    for q in range(NSLOT):
        fire(q, q)

    def tbody(t, carry):
        c0 = NSLOT * t
        for q in range(NSLOT):
            drain(q)
            compute(c0 + q, q)
            fire(c0 + q + NSLOT, q)
        return carry

    lax.fori_loop(0, NCH // NSLOT, tbody, 0)

    for q in range(NCH % NSLOT):
        drain(q)
        compute((NCH // NSLOT) * NSLOT + q, q)

    pltpu.sync_copy(res_v, out_hbm.at[wid])


@jax.jit
def _mf(user_indices, item_indices, user_embedding, item_embedding,
        user_bias, item_bias):
    uidx = user_indices.astype(jnp.int32).reshape(NW, NCH, CH)
    iidx = item_indices.astype(jnp.int32).reshape(NW, NCH, CH)
    ut = user_embedding.reshape(-1, 8, D)
    it = item_embedding.reshape(-1, 8, D)
    ub = user_bias.reshape(-1)
    ib = item_bias.reshape(-1)

    mesh = plsc.VectorSubcoreMesh(core_axis_name="c", subcore_axis_name="s")
    run = pl.kernel(
        _mf_body,
        out_type=jax.ShapeDtypeStruct((NW, BPW), jnp.float32),
        mesh=mesh,
        compiler_params=pltpu.CompilerParams(
            needs_layout_passes=False, use_tc_tiling_on_sc=True),
        scratch_types=[
            pltpu.VMEM((NCH, CH), jnp.int32),
            pltpu.VMEM((NCH, CH), jnp.int32),
            pltpu.VMEM((NCH, CH), jnp.int32),
            pltpu.VMEM((NCH, CH), jnp.int32),
            pltpu.VMEM((3, CH, 8, D), jnp.float32),
            pltpu.VMEM((3, CH, 8, D), jnp.float32),
            pltpu.VMEM((3, CH), jnp.float32),
            pltpu.VMEM((3, CH), jnp.float32),
            pltpu.VMEM((BPW,), jnp.float32),
            pltpu.SemaphoreType.DMA,
            pltpu.SemaphoreType.DMA,
            pltpu.SemaphoreType.DMA,
            pltpu.SemaphoreType.DMA,
            pltpu.SemaphoreType.DMA,
            pltpu.SemaphoreType.DMA,
        ],
    )
    out = run(uidx, iidx, ut, it, ub, ib)
    return out.reshape(B)


def kernel(user_indices, item_indices, user_embedding, item_embedding,
           user_bias, item_bias):
    return _mf(user_indices, item_indices, user_embedding, item_embedding,
               user_bias, item_bias)
